# baseline (speedup 1.0000x reference)
import functools

import jax
import jax.numpy as jnp
from jax import lax
from jax.experimental import pallas as pl
from jax.experimental.pallas import tpu as pltpu
from jax.experimental.pallas import tpu_sc as plsc

N_INTERVAL = 100
K_KEYPOINTS = 100000
B = 16384
D = 3

NC = 2
NS = 16
L = 16
NW = NC * NS
BPW = B // NW
WORDS = BPW * D
G = 128
NG = BPW // G

_mesh = plsc.VectorSubcoreMesh(
    core_axis_name="c", subcore_axis_name="s", num_cores=NC, num_subcores=NS
)


@functools.partial(
    pl.kernel,
    out_type=jax.ShapeDtypeStruct((B * D,), jnp.float32),
    mesh=_mesh,
    compiler_params=pltpu.CompilerParams(
        needs_layout_passes=False, use_tc_tiling_on_sc=False),
    scratch_types=[
        pltpu.VMEM((BPW,), jnp.int32),
        pltpu.VMEM((WORDS,), jnp.float32),
        pltpu.VMEM((BPW,), jnp.float32),
        pltpu.VMEM((BPW,), jnp.float32),
        pltpu.VMEM((BPW,), jnp.int32),
        pltpu.VMEM((BPW,), jnp.int32),
        pltpu.VMEM((BPW, D), jnp.float32),
        pltpu.VMEM((BPW, D), jnp.float32),
        pltpu.VMEM((WORDS,), jnp.float32),
        pltpu.SemaphoreType.DMA,
        pltpu.SemaphoreType.DMA,
    ],
)
def _position_sc(x_hbm, i_hbm, delta_hbm, out_hbm,
                 i_v, x_v, wl_v, wr_v, idxl_v, idxr_v, dl_v, dr_v, out_v,
                 sem_x, sem_g):
    wid = lax.axis_index("s") * NC + lax.axis_index("c")
    base = wid * BPW

    pltpu.sync_copy(i_hbm.at[pl.ds(base, BPW)], i_v)
    x_cp = pltpu.async_copy(x_hbm.at[pl.ds(base * D, WORDS)], x_v, sem_x)

    vn_i = jnp.full((L,), N_INTERVAL, jnp.int32)
    vn_f = jnp.full((L,), float(N_INTERVAL), jnp.float32)
    v1_i = jnp.full((L,), 1, jnp.int32)
    v1_f = jnp.full((L,), 1.0, jnp.float32)
    for s in range(BPW // L):
        iv = i_v[pl.ds(s * L, L)]
        left = lax.div(iv, vn_i)
        raw = lax.div(iv.astype(jnp.float32), vn_f)
        leftf = left.astype(jnp.float32)
        wl_v[pl.ds(s * L, L)] = leftf + v1_f - raw
        wr_v[pl.ds(s * L, L)] = raw - leftf
        idxl_v[pl.ds(s * L, L)] = left
        idxr_v[pl.ds(s * L, L)] = left + v1_i

    cps = []
    for j in range(NG):
        cps.append(pltpu.async_copy(
            delta_hbm.at[idxl_v.at[pl.ds(j * G, G)]],
            dl_v.at[pl.ds(j * G, G)], sem_g))
        cps.append(pltpu.async_copy(
            delta_hbm.at[idxr_v.at[pl.ds(j * G, G)]],
            dr_v.at[pl.ds(j * G, G)], sem_g))
    x_cp.wait()
    for cp in cps:
        cp.wait()

    iota = lax.iota(jnp.int32, L)
    vD = jnp.full((L,), D, jnp.int32)
    rows = [lax.div(iota + jnp.full((L,), L * c, jnp.int32), vD)
            for c in range(D)]
    cols = [lax.rem(iota + jnp.full((L,), L * c, jnp.int32), vD)
            for c in range(D)]
    for u in range(BPW // L):
        rbase = jnp.full((L,), u * L, jnp.int32)
        for c in range(D):
            t = u * D + c
            row = rows[c] + rbase
            col = cols[c]
            wl = plsc.load_gather(wl_v, [row])
            wr = plsc.load_gather(wr_v, [row])
            dl = plsc.load_gather(dl_v, [row, col])
            dr = plsc.load_gather(dr_v, [row, col])
            xv = x_v[pl.ds(t * L, L)]
            out_v[pl.ds(t * L, L)] = xv + wl * dl + wr * dr

    pltpu.sync_copy(out_v, out_hbm.at[pl.ds(base * D, WORDS)])


def kernel(x, i, delta):
    out_flat = _position_sc(x.reshape(-1), i, delta)
    return out_flat.reshape(B, D)


# speedup vs baseline: 4.4443x; 4.4443x over previous
import functools

import jax
import jax.numpy as jnp
from jax import lax
from jax.experimental import pallas as pl
from jax.experimental.pallas import tpu as pltpu
from jax.experimental.pallas import tpu_sc as plsc

N_INTERVAL = 100
K_KEYPOINTS = 100000
B = 16384
D = 3

NC = 2
NS = 16
L = 16
NW = NC * NS
BPW = B // NW
G = 128
NG = BPW // G
NR = 2 * D
SPB = G // L

_mesh = plsc.VectorSubcoreMesh(
    core_axis_name="c", subcore_axis_name="s", num_cores=NC, num_subcores=NS
)


@functools.partial(
    pl.kernel,
    out_type=jax.ShapeDtypeStruct((D * B,), jnp.float32),
    mesh=_mesh,
    compiler_params=pltpu.CompilerParams(
        needs_layout_passes=False, use_tc_tiling_on_sc=False),
    scratch_types=[
        pltpu.VMEM((BPW,), jnp.int32),
        pltpu.VMEM((D, BPW), jnp.float32),
        pltpu.VMEM((BPW,), jnp.float32),
        pltpu.VMEM((BPW,), jnp.float32),
        pltpu.VMEM((NR * NG, G), jnp.int32),
        pltpu.VMEM((NR * NG, G), jnp.float32),
        pltpu.VMEM((D, BPW), jnp.float32),
        pltpu.SemaphoreType.DMA,
        pltpu.SemaphoreType.DMA,
    ],
)
def _position_sc(x_hbm, i_hbm, delta_hbm, out_hbm,
                 i_v, x_v, wl_v, wr_v, idx_v, d_v, out_v, sem_x, sem_g):
    wid = lax.axis_index("s") * NC + lax.axis_index("c")
    base = wid * BPW

    pltpu.sync_copy(i_hbm.at[pl.ds(base, BPW)], i_v)
    x_cps = [
        pltpu.async_copy(x_hbm.at[pl.ds(c * B + base, BPW)], x_v.at[c], sem_x)
        for c in range(D)
    ]

    vn_i = jnp.full((L,), N_INTERVAL, jnp.int32)
    vn_f = jnp.full((L,), float(N_INTERVAL), jnp.float32)
    v1_f = jnp.full((L,), 1.0, jnp.float32)
    offs = [jnp.full((L,), c * K_KEYPOINTS + s, jnp.int32)
            for c in range(D) for s in range(2)]
    g_cps = []
    for blk in range(NG):
        for sub in range(SPB):
            s = blk * SPB + sub
            iv = i_v[pl.ds(s * L, L)]
            left = lax.div(iv, vn_i)
            raw = lax.div(iv.astype(jnp.float32), vn_f)
            leftf = left.astype(jnp.float32)
            wl_v[pl.ds(s * L, L)] = leftf + v1_f - raw
            wr_v[pl.ds(s * L, L)] = raw - leftf
            for r in range(NR):
                idx_v[r * NG + blk, pl.ds(sub * L, L)] = left + offs[r]
        for r in range(NR):
            row = r * NG + blk
            g_cps.append(pltpu.async_copy(
                delta_hbm.at[idx_v.at[row]], d_v.at[row], sem_g))
    for cp in x_cps:
        cp.wait()
    for cp in g_cps:
        cp.wait()

    for s in range(BPW // L):
        wl = wl_v[pl.ds(s * L, L)]
        wr = wr_v[pl.ds(s * L, L)]
        blk, col = s // SPB, (s % SPB) * L
        for c in range(D):
            dl = d_v[(2 * c) * NG + blk, pl.ds(col, L)]
            dr = d_v[(2 * c + 1) * NG + blk, pl.ds(col, L)]
            xc = x_v[c, pl.ds(s * L, L)]
            out_v[c, pl.ds(s * L, L)] = xc + wl * dl + wr * dr

    for c in range(D):
        pltpu.sync_copy(out_v.at[c], out_hbm.at[pl.ds(c * B + base, BPW)])


def kernel(x, i, delta):
    out_flat = _position_sc(
        x.T.reshape(-1), i, delta.T.reshape(-1))
    return out_flat.reshape(D, B).T


# speedup vs baseline: 5.0008x; 1.1252x over previous
import functools

import jax
import jax.numpy as jnp
from jax import lax
from jax.experimental import pallas as pl
from jax.experimental.pallas import tpu as pltpu
from jax.experimental.pallas import tpu_sc as plsc

N_INTERVAL = 100
K_KEYPOINTS = 100000
B = 16384
D = 3

NC = 2
NS = 16
L = 16
NW = NC * NS
BPW = B // NW
G = 128
NG = BPW // G
NR = 2 * D
SPB = G // L

_mesh = plsc.VectorSubcoreMesh(
    core_axis_name="c", subcore_axis_name="s", num_cores=NC, num_subcores=NS
)


@functools.partial(
    pl.kernel,
    out_type=jax.ShapeDtypeStruct((D * B,), jnp.float32),
    mesh=_mesh,
    compiler_params=pltpu.CompilerParams(
        needs_layout_passes=False, use_tc_tiling_on_sc=False),
    scratch_types=[
        pltpu.VMEM((BPW,), jnp.int32),
        pltpu.VMEM((D, BPW), jnp.float32),
        pltpu.VMEM((BPW,), jnp.float32),
        pltpu.VMEM((BPW,), jnp.float32),
        pltpu.VMEM((NR * NG, G), jnp.int32),
        pltpu.VMEM((NR * NG, G), jnp.float32),
        pltpu.VMEM((D, BPW), jnp.float32),
        pltpu.SemaphoreType.DMA,
        pltpu.SemaphoreType.DMA,
    ],
)
def _position_sc(x_hbm, i_hbm, delta_hbm, out_hbm,
                 i_v, x_v, wl_v, wr_v, idx_v, d_v, out_v, sem_x, sem_g):
    wid = lax.axis_index("s") * NC + lax.axis_index("c")
    base = wid * BPW

    pltpu.sync_copy(i_hbm.at[pl.ds(base, BPW)], i_v)
    x_cps = [
        pltpu.async_copy(x_hbm.at[pl.ds(c * B + base, BPW)], x_v.at[c], sem_x)
        for c in range(D)
    ]

    vinv = jnp.full((L,), 1.0 / N_INTERVAL, jnp.float32)
    vmax_i = jnp.full((L,), K_KEYPOINTS - 2, jnp.int32)
    v1_f = jnp.full((L,), 1.0, jnp.float32)
    offs = [jnp.full((L,), c * K_KEYPOINTS + s, jnp.int32)
            for c in range(D) for s in range(2)]
    g_cps = []
    for blk in range(NG):
        for sub in range(SPB):
            s = blk * SPB + sub
            iv = i_v[pl.ds(s * L, L)]
            raw = iv.astype(jnp.float32) * vinv
            left = jnp.minimum(raw.astype(jnp.int32), vmax_i)
            leftf = left.astype(jnp.float32)
            wl_v[pl.ds(s * L, L)] = leftf + v1_f - raw
            wr_v[pl.ds(s * L, L)] = raw - leftf
            for r in range(NR):
                idx_v[r * NG + blk, pl.ds(sub * L, L)] = left + offs[r]
        for r in range(NR):
            row = r * NG + blk
            g_cps.append(pltpu.async_copy(
                delta_hbm.at[idx_v.at[row]], d_v.at[row], sem_g))
    for cp in x_cps:
        cp.wait()
    for cp in g_cps:
        cp.wait()

    for s in range(BPW // L):
        wl = wl_v[pl.ds(s * L, L)]
        wr = wr_v[pl.ds(s * L, L)]
        blk, col = s // SPB, (s % SPB) * L
        for c in range(D):
            dl = d_v[(2 * c) * NG + blk, pl.ds(col, L)]
            dr = d_v[(2 * c + 1) * NG + blk, pl.ds(col, L)]
            xc = x_v[c, pl.ds(s * L, L)]
            out_v[c, pl.ds(s * L, L)] = xc + wl * dl + wr * dr

    for c in range(D):
        pltpu.sync_copy(out_v.at[c], out_hbm.at[pl.ds(c * B + base, BPW)])


def kernel(x, i, delta):
    out_flat = _position_sc(
        x.T.reshape(-1), i, delta.T.reshape(-1))
    return out_flat.reshape(D, B).T


# speedup vs baseline: 5.0367x; 1.0072x over previous
import functools

import jax
import jax.numpy as jnp
from jax import lax
from jax.experimental import pallas as pl
from jax.experimental.pallas import tpu as pltpu
from jax.experimental.pallas import tpu_sc as plsc

N_INTERVAL = 100
K_KEYPOINTS = 100000
B = 16384
D = 3

NC = 2
NS = 16
L = 16
NW = NC * NS
BPW = B // NW
G = 128
NG = BPW // G
NR = 2 * D
SPB = G // L

_mesh = plsc.VectorSubcoreMesh(
    core_axis_name="c", subcore_axis_name="s", num_cores=NC, num_subcores=NS
)


@functools.partial(
    pl.kernel,
    out_type=jax.ShapeDtypeStruct((D * B,), jnp.float32),
    mesh=_mesh,
    compiler_params=pltpu.CompilerParams(
        needs_layout_passes=False, use_tc_tiling_on_sc=False),
    scratch_types=[
        pltpu.VMEM((BPW,), jnp.int32),
        pltpu.VMEM((D, BPW), jnp.float32),
        pltpu.VMEM((BPW,), jnp.float32),
        pltpu.VMEM((BPW,), jnp.float32),
        pltpu.VMEM((NR, BPW), jnp.int32),
        pltpu.VMEM((NR, BPW), jnp.float32),
        pltpu.VMEM((D, BPW), jnp.float32),
        pltpu.SemaphoreType.DMA,
        pltpu.SemaphoreType.DMA,
    ],
)
def _position_sc(x_hbm, i_hbm, delta_hbm, out_hbm,
                 i_v, x_v, wl_v, wr_v, idx_v, d_v, out_v, sem_x, sem_g):
    wid = lax.axis_index("s") * NC + lax.axis_index("c")
    base = wid * BPW

    pltpu.sync_copy(i_hbm.at[pl.ds(base, BPW)], i_v)
    x_cps = [
        pltpu.async_copy(x_hbm.at[pl.ds(c * B + base, BPW)], x_v.at[c], sem_x)
        for c in range(D)
    ]

    vinv = jnp.full((L,), 1.0 / N_INTERVAL, jnp.float32)
    vmax_i = jnp.full((L,), K_KEYPOINTS - 2, jnp.int32)
    v1_f = jnp.full((L,), 1.0, jnp.float32)
    offs = [jnp.full((L,), c * K_KEYPOINTS + s, jnp.int32)
            for c in range(D) for s in range(2)]
    for s in range(BPW // L):
        iv = i_v[pl.ds(s * L, L)]
        raw = iv.astype(jnp.float32) * vinv
        left = jnp.minimum(raw.astype(jnp.int32), vmax_i)
        leftf = left.astype(jnp.float32)
        wl_v[pl.ds(s * L, L)] = leftf + v1_f - raw
        wr_v[pl.ds(s * L, L)] = raw - leftf
        for r in range(NR):
            idx_v[r, pl.ds(s * L, L)] = left + offs[r]
    g_cps = [
        pltpu.async_copy(delta_hbm.at[idx_v.at[r]], d_v.at[r], sem_g)
        for r in range(NR)
    ]
    for cp in x_cps:
        cp.wait()
    for cp in g_cps:
        cp.wait()

    for s in range(BPW // L):
        wl = wl_v[pl.ds(s * L, L)]
        wr = wr_v[pl.ds(s * L, L)]
        for c in range(D):
            dl = d_v[2 * c, pl.ds(s * L, L)]
            dr = d_v[2 * c + 1, pl.ds(s * L, L)]
            xc = x_v[c, pl.ds(s * L, L)]
            out_v[c, pl.ds(s * L, L)] = xc + wl * dl + wr * dr

    o_cps = [
        pltpu.async_copy(out_v.at[c], out_hbm.at[pl.ds(c * B + base, BPW)],
                         sem_x)
        for c in range(D)
    ]
    for cp in o_cps:
        cp.wait()


def kernel(x, i, delta):
    out_flat = _position_sc(
        x.T.reshape(-1), i, delta.T.reshape(-1))
    return out_flat.reshape(D, B).T
